# baseline (speedup 1.0000x reference)
import jax
import jax.numpy as jnp
from jax.experimental import pallas as pl


def _concat_copy_kernel(a_ref, b_ref, o_ref):
    s = pl.program_id(1)

    @pl.when(s == 0)
    def _():
        o_ref[...] = a_ref[...][:, None]

    @pl.when(s == 1)
    def _():
        o_ref[...] = b_ref[...][:, None]


def kernel(a, b, scatter_a, scatter_b):
    B, C, H, W = a.shape
    nH = 8
    hb = H // nH

    out5 = pl.pallas_call(
        _concat_copy_kernel,
        grid=(B, 2, nH),
        in_specs=[
            pl.BlockSpec((1, C, hb, W),
                         lambda bi, s, h: (bi, 0, h * (1 - s) + (nH - 1) * s, 0)),
            pl.BlockSpec((1, C, hb, W),
                         lambda bi, s, h: (bi, 0, h * s, 0)),
        ],
        out_specs=pl.BlockSpec((1, 1, C, hb, W),
                               lambda bi, s, h: (bi, s, 0, h, 0)),
        out_shape=jax.ShapeDtypeStruct((B, 2, C, H, W), a.dtype),
    )(a, b)
    return out5.reshape(B, 2 * C, H, W)


# speedup vs baseline: 1.4907x; 1.4907x over previous
import jax
import jax.numpy as jnp
from jax.experimental import pallas as pl


def _concat_copy_kernel(a_ref, b_ref, o_ref):
    s = pl.program_id(1)

    @pl.when(s == 0)
    def _():
        o_ref[...] = a_ref[...][:, None]

    @pl.when(s == 1)
    def _():
        o_ref[...] = b_ref[...][:, None]


def kernel(a, b, scatter_a, scatter_b):
    B, C, H, W = a.shape
    nH = 2
    hb = H // nH

    out5 = pl.pallas_call(
        _concat_copy_kernel,
        grid=(B, 2, nH),
        in_specs=[
            pl.BlockSpec((1, C, hb, W),
                         lambda bi, s, h: (bi, 0, h * (1 - s) + (nH - 1) * s, 0)),
            pl.BlockSpec((1, C, hb, W),
                         lambda bi, s, h: (bi, 0, h * s, 0)),
        ],
        out_specs=pl.BlockSpec((1, 1, C, hb, W),
                               lambda bi, s, h: (bi, s, 0, h, 0)),
        out_shape=jax.ShapeDtypeStruct((B, 2, C, H, W), a.dtype),
    )(a, b)
    return out5.reshape(B, 2 * C, H, W)
